# Initial kernel scaffold; baseline (speedup 1.0000x reference)
import jax
import jax.numpy as jnp
from jax.experimental import pallas as pl


def kernel(x, enc_params, dec_params, lin_W, lin_b):
    raise NotImplementedError("write your pallas kernel here")


# speedup vs baseline: 1.2375x; 1.2375x over previous
import jax
import jax.numpy as jnp
from jax.experimental import pallas as pl
from jax.experimental.pallas import tpu as pltpu

N = 128
D = 16
LATENT = 8
HE = 64
OE = 32
ALPHA = 0.2
B = 32
NPW = 12


def _leaky(v):
    return jnp.where(v >= 0, v, ALPHA * v)


def _mp_layer(x, wa, wb, c, b1, w2, b2, nws):
    sq = jnp.sum(x * x, axis=1, keepdims=True)
    g = jax.lax.dot_general(x, x, (((1,), (1,)), ((), ())),
                            preferred_element_type=jnp.float32)
    dist = sq + sq.T - 2.0 * g
    a = jnp.dot(x, wa, preferred_element_type=jnp.float32) + b1
    bm = jnp.dot(x, wb, preferred_element_type=jnp.float32)
    h1 = _leaky(a[:, None, :] + bm[None, :, :]
                + dist[:, :, None] * c[None, :, :])
    h2 = _leaky(
        jnp.dot(h1.reshape(N * N, HE), w2,
                preferred_element_type=jnp.float32) + b2)
    w = jnp.exp(-dist)
    agg = jnp.sum(w[:, :, None] * h2.reshape(N, N, OE), axis=1)
    h = jnp.concatenate([x, agg], axis=-1)
    for i, (nw, nb) in enumerate(nws):
        h = jnp.dot(h, nw, preferred_element_type=jnp.float32) + nb
        if i < len(nws) - 1:
            h = _leaky(h)
    return h


def _body(x_ref, *refs):
    refs = list(refs)
    latent_ref, y_ref = refs[-2], refs[-1]
    wrefs = refs[:-2]

    def mp_args(k):
        base = wrefs[k * NPW:(k + 1) * NPW]
        wa, wb, c, b1, w2, b2 = (r[...] for r in base[:6])
        nws = [(base[6 + 2 * i][...], base[7 + 2 * i][...]) for i in range(3)]
        return wa, wb, c, b1, w2, b2, nws

    wr_ref, br_ref = wrefs[4 * NPW], wrefs[4 * NPW + 1]

    x = x_ref[0]
    z = _mp_layer(x, *mp_args(0))
    z = _mp_layer(z, *mp_args(1))
    lat = jnp.sum(z, axis=0, keepdims=True)
    latent_ref[0] = lat
    y0 = br_ref[...]
    for k in range(LATENT):
        y0 = y0 + wr_ref[k] * lat[0:1, k:k + 1]
    y = _mp_layer(y0, *mp_args(2))
    y = _mp_layer(y, *mp_args(3))
    y_ref[0] = y


def _flatten_mp(p, din):
    (w1, b1), (w2, b2) = p["edge"]
    arrs = [w1[:din], w1[din:2 * din], w1[2 * din:2 * din + 1],
            b1.reshape(1, HE), w2, b2.reshape(1, OE)]
    for (w, bb) in p["node"]:
        arrs += [w, bb.reshape(1, -1)]
    return arrs


def kernel(x, enc_params, dec_params, lin_W, lin_b):
    ops = []
    ops += _flatten_mp(enc_params[0], D)
    ops += _flatten_mp(enc_params[1], D)
    ops += _flatten_mp(dec_params[0], LATENT)
    ops += _flatten_mp(dec_params[1], LATENT)
    ops.append(lin_W.reshape(LATENT, N, LATENT))
    ops.append(lin_b.reshape(N, LATENT))

    def const_spec(a):
        nd = a.ndim
        return pl.BlockSpec(a.shape, lambda b, _n=nd: (0,) * _n)

    in_specs = [pl.BlockSpec((1, N, D), lambda b: (b, 0, 0))]
    in_specs += [const_spec(a) for a in ops]

    latent, y = pl.pallas_call(
        _body,
        grid=(B,),
        in_specs=in_specs,
        out_specs=[
            pl.BlockSpec((1, 1, LATENT), lambda b: (b, 0, 0)),
            pl.BlockSpec((1, N, D), lambda b: (b, 0, 0)),
        ],
        out_shape=[
            jax.ShapeDtypeStruct((B, 1, LATENT), jnp.float32),
            jax.ShapeDtypeStruct((B, N, D), jnp.float32),
        ],
        compiler_params=pltpu.CompilerParams(
            dimension_semantics=("parallel",)),
    )(x, *ops)
    return latent.reshape(1, B, LATENT), y


# speedup vs baseline: 2.2698x; 1.8342x over previous
import jax
import jax.numpy as jnp
from jax.experimental import pallas as pl
from jax.experimental.pallas import tpu as pltpu

N = 128
D = 16
LATENT = 8
HE = 64
OE = 32
ALPHA = 0.2
B = 32
P = 4
KP = N // P
NPW = 13


def _leaky(v):
    return jnp.maximum(v, ALPHA * v)


def _mm(a, b):
    return jax.lax.dot_general(a, b, (((a.ndim - 1,), (0,)), ((), ())),
                               preferred_element_type=jnp.float32)


def _mp_layer(x, wa, wb, c, b1, c4, w2p, b2p, nws, e4, m4, q):
    sq = jnp.sum(x * x, axis=1, keepdims=True)
    g = jax.lax.dot_general(x, x, (((1,), (1,)), ((), ())),
                            preferred_element_type=jnp.float32)
    dist = sq + sq.T - 2.0 * g
    w = jnp.exp(-dist)
    sqc = sq * c
    a2 = _mm(x, wa) + b1 + sqc
    bm2 = _mm(x, wb) + sqc
    at = jnp.concatenate([a2] * P, axis=1)
    bmt = jnp.concatenate([bm2] * P, axis=1) * m4
    bm4 = _mm(q, bmt)
    gp = jnp.stack([g[:, P * k:P * k + P] for k in range(KP)], axis=0)
    wp = jnp.stack([w[:, P * k:P * k + P] for k in range(KP)], axis=0)
    t = _mm(gp, c4)
    h1 = _leaky(at[None, :, :] + bm4[:, None, :] + t)
    h2 = _leaky(_mm(h1, w2p) + b2p)
    w4 = _mm(wp, e4)
    s1 = jnp.sum(h2 * w4, axis=0)
    agg = (s1[:, :OE] + s1[:, OE:2 * OE]
           + s1[:, 2 * OE:3 * OE] + s1[:, 3 * OE:])
    h = jnp.concatenate([x, agg], axis=-1)
    for i, (nw, nb) in enumerate(nws):
        h = _mm(h, nw) + nb
        if i < len(nws) - 1:
            h = _leaky(h)
    return h


def _body(x_ref, e4_ref, m4_ref, q_ref, *refs):
    refs = list(refs)
    latent_ref, y_ref = refs[-2], refs[-1]
    wrefs = refs[:-2]
    consts = (e4_ref[...], m4_ref[...], q_ref[...])

    def mp_args(k):
        base = wrefs[k * NPW:(k + 1) * NPW]
        wa, wb, c, b1, c4, w2p, b2p = (r[...] for r in base[:7])
        nws = [(base[7 + 2 * i][...], base[8 + 2 * i][...]) for i in range(3)]
        return wa, wb, c, b1, c4, w2p, b2p, nws

    wr_ref, br_ref = wrefs[4 * NPW], wrefs[4 * NPW + 1]

    x = x_ref[0]
    z = _mp_layer(x, *mp_args(0), *consts)
    z = _mp_layer(z, *mp_args(1), *consts)
    lat = jnp.sum(z, axis=0, keepdims=True)
    latent_ref[0] = lat
    y0 = br_ref[...]
    for k in range(LATENT):
        y0 = y0 + wr_ref[k] * lat[0:1, k:k + 1]
    y = _mp_layer(y0, *mp_args(2), *consts)
    y = _mp_layer(y, *mp_args(3), *consts)
    y_ref[0] = y


def _blockdiag(m):
    z = jnp.zeros_like(m)
    return jnp.concatenate(
        [jnp.concatenate([m if mm == k else z for mm in range(P)], axis=1)
         for k in range(P)], axis=0)


def _flatten_mp(p, din):
    (w1, b1), (w2, b2) = p["edge"]
    c = w1[2 * din:2 * din + 1]
    arrs = [w1[:din], w1[din:2 * din], c, b1.reshape(1, HE),
            _blockdiag(-2.0 * c), _blockdiag(w2),
            jnp.concatenate([b2.reshape(1, OE)] * P, axis=1)]
    for (w, bb) in p["node"]:
        arrs += [w, bb.reshape(1, -1)]
    return arrs


def kernel(x, enc_params, dec_params, lin_W, lin_b):
    e4 = _blockdiag(jnp.ones((1, OE), jnp.float32))
    m4 = jnp.kron(jnp.tile(jnp.eye(P, dtype=jnp.float32), (KP, 1)),
                  jnp.ones((1, HE), jnp.float32))
    q = jnp.kron(jnp.eye(KP, dtype=jnp.float32),
                 jnp.ones((1, P), jnp.float32))
    ops = [e4, m4, q]
    ops += _flatten_mp(enc_params[0], D)
    ops += _flatten_mp(enc_params[1], D)
    ops += _flatten_mp(dec_params[0], LATENT)
    ops += _flatten_mp(dec_params[1], LATENT)
    ops.append(lin_W.reshape(LATENT, N, LATENT))
    ops.append(lin_b.reshape(N, LATENT))

    def const_spec(a):
        nd = a.ndim
        return pl.BlockSpec(a.shape, lambda b, _n=nd: (0,) * _n)

    in_specs = [pl.BlockSpec((1, N, D), lambda b: (b, 0, 0))]
    in_specs += [const_spec(a) for a in ops]

    latent, y = pl.pallas_call(
        _body,
        grid=(B,),
        in_specs=in_specs,
        out_specs=[
            pl.BlockSpec((1, 1, LATENT), lambda b: (b, 0, 0)),
            pl.BlockSpec((1, N, D), lambda b: (b, 0, 0)),
        ],
        out_shape=[
            jax.ShapeDtypeStruct((B, 1, LATENT), jnp.float32),
            jax.ShapeDtypeStruct((B, N, D), jnp.float32),
        ],
        compiler_params=pltpu.CompilerParams(
            dimension_semantics=("parallel",)),
    )(x, *ops)
    return latent.reshape(1, B, LATENT), y


# speedup vs baseline: 2.4566x; 1.0823x over previous
import jax
import jax.numpy as jnp
from jax.experimental import pallas as pl
from jax.experimental.pallas import tpu as pltpu

N = 128
D = 16
LATENT = 8
HE = 64
OE = 32
ALPHA = 0.2
B = 32
P = 4
KP = N // P
PHE = P * HE
POE = P * OE
NPW = 10
BF = jnp.bfloat16


def _leaky(v):
    return jnp.maximum(v, ALPHA * v)


def _mm(a, b):
    return jax.lax.dot_general(a, b, (((a.ndim - 1,), (0,)), ((), ())),
                               preferred_element_type=jnp.float32)


def _mp_layer(x, wb, tw, w2p, b2p, nws, e32, sall):
    sq = jnp.sum(x * x, axis=1, keepdims=True)
    g = jax.lax.dot_general(x, x, (((1,), (1,)), ((), ())),
                            preferred_element_type=jnp.float32)
    dist = sq + sq.T - 2.0 * g
    w = jnp.exp(-dist)
    xbf = x.astype(BF)
    bm2 = _mm(xbf, wb)
    bm4p = _mm(sall, bm2.astype(BF))
    bm4 = jnp.concatenate(
        [bm4p[KP * m:KP * (m + 1)] for m in range(P)], axis=1).astype(BF)
    lhs = jnp.concatenate([dist.astype(BF), xbf, jnp.ones((N, 1), BF)],
                          axis=1)
    t_all = jax.lax.dot_general(lhs, tw, (((1,), (0,)), ((), ())),
                                preferred_element_type=jnp.float32)
    t_all = t_all.astype(BF)
    w4_all = jax.lax.dot_general(w.astype(BF), e32, (((1,), (0,)), ((), ())),
                                 preferred_element_type=jnp.float32)
    acc = [jnp.zeros((N, POE), jnp.float32) for _ in range(4)]
    for k in range(KP):
        h1k = _leaky(bm4[k:k + 1, :] + t_all[:, PHE * k:PHE * (k + 1)])
        h2k = _leaky(
            jax.lax.dot_general(h1k, w2p, (((1,), (0,)), ((), ())),
                                preferred_element_type=jnp.float32) + b2p)
        acc[k % 4] = acc[k % 4] + h2k * w4_all[:, POE * k:POE * (k + 1)]
    s1 = (acc[0] + acc[1]) + (acc[2] + acc[3])
    agg = (s1[:, :OE] + s1[:, OE:2 * OE]
           + s1[:, 2 * OE:3 * OE] + s1[:, 3 * OE:])
    h = jnp.concatenate([x, agg], axis=-1)
    for i, (nw, nb) in enumerate(nws):
        h = _mm(h, nw) + nb
        if i < len(nws) - 1:
            h = _leaky(h)
    return h


def _body(x_ref, e32_ref, sall_ref, *refs):
    refs = list(refs)
    latent_ref, y_ref = refs[-2], refs[-1]
    wrefs = refs[:-2]
    consts = (e32_ref[...], sall_ref[...])

    def mp_args(k):
        base = wrefs[k * NPW:(k + 1) * NPW]
        wb, tw, w2p, b2p = (r[...] for r in base[:4])
        nws = [(base[4 + 2 * i][...], base[5 + 2 * i][...]) for i in range(3)]
        return wb, tw, w2p, b2p, nws

    wr_ref, br_ref = wrefs[4 * NPW], wrefs[4 * NPW + 1]

    x = x_ref[0]
    z = _mp_layer(x, *mp_args(0), *consts)
    z = _mp_layer(z, *mp_args(1), *consts)
    lat = jnp.sum(z, axis=0, keepdims=True)
    latent_ref[0] = lat
    y0t = br_ref[...]
    for k in range(LATENT):
        y0t = y0t + wr_ref[k] * lat[0:1, k:k + 1]
    y0 = y0t.T
    y = _mp_layer(y0, *mp_args(2), *consts)
    y = _mp_layer(y, *mp_args(3), *consts)
    y_ref[0] = y


def _blockdiag(m):
    z = jnp.zeros_like(m)
    return jnp.concatenate(
        [jnp.concatenate([m if mm == k else z for mm in range(P)], axis=1)
         for k in range(P)], axis=0)


def _flatten_mp(p, din):
    (w1, b1), (w2, b2) = p["edge"]
    c = w1[2 * din:2 * din + 1]
    tw = jnp.concatenate(
        [jnp.kron(jnp.eye(N, dtype=jnp.float32), c),
         jnp.tile(w1[:din], (1, N)),
         jnp.tile(b1.reshape(1, HE), (1, N))], axis=0).astype(BF)
    arrs = [w1[din:2 * din].astype(BF), tw,
            _blockdiag(w2).astype(BF),
            jnp.concatenate([b2.reshape(1, OE)] * P, axis=1)]
    for (w, bb) in p["node"]:
        arrs += [w, bb.reshape(1, -1)]
    return arrs


def kernel(x, enc_params, dec_params, lin_W, lin_b):
    f32 = jnp.float32
    e32 = jnp.kron(jnp.eye(N, dtype=f32),
                   jnp.ones((1, OE), f32)).astype(BF)
    r = jnp.arange(N)
    sall = jax.nn.one_hot(P * (r % KP) + r // KP, N, dtype=BF)
    ops = [e32, sall]
    ops += _flatten_mp(enc_params[0], D)
    ops += _flatten_mp(enc_params[1], D)
    ops += _flatten_mp(dec_params[0], LATENT)
    ops += _flatten_mp(dec_params[1], LATENT)
    ops.append(jnp.transpose(lin_W.reshape(LATENT, N, LATENT), (0, 2, 1)))
    ops.append(lin_b.reshape(N, LATENT).T)

    def const_spec(a):
        nd = a.ndim
        return pl.BlockSpec(a.shape, lambda b, _n=nd: (0,) * _n)

    in_specs = [pl.BlockSpec((1, N, D), lambda b: (b, 0, 0))]
    in_specs += [const_spec(a) for a in ops]

    latent, y = pl.pallas_call(
        _body,
        grid=(B,),
        in_specs=in_specs,
        out_specs=[
            pl.BlockSpec((1, 1, LATENT), lambda b: (b, 0, 0)),
            pl.BlockSpec((1, N, D), lambda b: (b, 0, 0)),
        ],
        out_shape=[
            jax.ShapeDtypeStruct((B, 1, LATENT), f32),
            jax.ShapeDtypeStruct((B, N, D), f32),
        ],
        compiler_params=pltpu.CompilerParams(
            dimension_semantics=("parallel",)),
    )(x, *ops)
    return latent.reshape(1, B, LATENT), y


# speedup vs baseline: 2.9338x; 1.1942x over previous
import jax
import jax.numpy as jnp
from jax.experimental import pallas as pl
from jax.experimental.pallas import tpu as pltpu

N = 128
D = 16
LATENT = 8
HE = 64
OE = 32
ALPHA = 0.2
B = 32
P = 4
KP = N // P
PHE = P * HE
POE = P * OE
NPW = 10
BF = jnp.bfloat16


def _leaky(v):
    return jnp.maximum(v, ALPHA * v)


def _mm(a, b):
    return jax.lax.dot_general(a, b, (((a.ndim - 1,), (0,)), ((), ())),
                               preferred_element_type=jnp.float32)


def _mp_layer(x2, wb, tw, w2p, b2p, nws, e32, sall):
    def _dist_w(xg):
        sq = jnp.sum(xg * xg, axis=1, keepdims=True)
        g = jax.lax.dot_general(xg, xg, (((1,), (1,)), ((), ())),
                                preferred_element_type=jnp.float32)
        dist = sq + sq.T - 2.0 * g
        return dist, jnp.exp(-dist)

    xa, xb = x2[:N], x2[N:]
    dist_a, w_a = _dist_w(xa)
    dist_b, w_b = _dist_w(xb)
    x2bf = x2.astype(BF)
    bm2 = _mm(x2bf, wb)

    def _bm4(bm2g):
        p4 = _mm(sall, bm2g.astype(BF))
        return jnp.concatenate(
            [p4[KP * m:KP * (m + 1)] for m in range(P)], axis=1).astype(BF)

    bm4_a = _bm4(bm2[:N])
    bm4_b = _bm4(bm2[N:])
    ones = jnp.ones((N, 1), BF)
    lhs = jnp.concatenate(
        [jnp.concatenate([dist_a.astype(BF), xa.astype(BF), ones], axis=1),
         jnp.concatenate([dist_b.astype(BF), xb.astype(BF), ones], axis=1)],
        axis=0)
    t_all = jax.lax.dot_general(lhs, tw, (((1,), (0,)), ((), ())),
                                preferred_element_type=jnp.float32)
    t_all = t_all.astype(BF)
    wcat = jnp.concatenate([w_a, w_b], axis=0).astype(BF)
    w4_all = jax.lax.dot_general(wcat, e32, (((1,), (0,)), ((), ())),
                                 preferred_element_type=jnp.float32)
    acc = [jnp.zeros((2 * N, POE), jnp.float32) for _ in range(4)]
    for k in range(KP):
        tk = t_all[:, PHE * k:PHE * (k + 1)]
        h1k = _leaky(jnp.concatenate(
            [tk[:N] + bm4_a[k:k + 1, :], tk[N:] + bm4_b[k:k + 1, :]],
            axis=0))
        h2k = _leaky(
            jax.lax.dot_general(h1k, w2p, (((1,), (0,)), ((), ())),
                                preferred_element_type=jnp.float32) + b2p)
        acc[k % 4] = acc[k % 4] + h2k * w4_all[:, POE * k:POE * (k + 1)]
    s1 = (acc[0] + acc[1]) + (acc[2] + acc[3])
    agg = (s1[:, :OE] + s1[:, OE:2 * OE]
           + s1[:, 2 * OE:3 * OE] + s1[:, 3 * OE:])
    h = jnp.concatenate([x2, agg], axis=-1)
    for i, (nw, nb) in enumerate(nws):
        h = _mm(h, nw) + nb
        if i < len(nws) - 1:
            h = _leaky(h)
    return h


def _body(x_ref, e32_ref, sall_ref, *refs):
    refs = list(refs)
    latent_ref, y_ref = refs[-2], refs[-1]
    wrefs = refs[:-2]
    consts = (e32_ref[...], sall_ref[...])

    def mp_args(k):
        base = wrefs[k * NPW:(k + 1) * NPW]
        wb, tw, w2p, b2p = (r[...] for r in base[:4])
        nws = [(base[4 + 2 * i][...], base[5 + 2 * i][...]) for i in range(3)]
        return wb, tw, w2p, b2p, nws

    wr_ref, br_ref = wrefs[4 * NPW], wrefs[4 * NPW + 1]

    x2 = x_ref[...].reshape(2 * N, D)
    z = _mp_layer(x2, *mp_args(0), *consts)
    z = _mp_layer(z, *mp_args(1), *consts)
    lat_a = jnp.sum(z[:N], axis=0, keepdims=True)
    lat_b = jnp.sum(z[N:], axis=0, keepdims=True)
    latent_ref[0] = lat_a
    latent_ref[1] = lat_b

    def _y0(lat):
        y0t = br_ref[...]
        for k in range(LATENT):
            y0t = y0t + wr_ref[k] * lat[0:1, k:k + 1]
        return y0t.T

    y0 = jnp.concatenate([_y0(lat_a), _y0(lat_b)], axis=0)
    y = _mp_layer(y0, *mp_args(2), *consts)
    y = _mp_layer(y, *mp_args(3), *consts)
    y_ref[...] = y.reshape(2, N, D)


def _blockdiag(m):
    z = jnp.zeros_like(m)
    return jnp.concatenate(
        [jnp.concatenate([m if mm == k else z for mm in range(P)], axis=1)
         for k in range(P)], axis=0)


def _flatten_mp(p, din):
    (w1, b1), (w2, b2) = p["edge"]
    c = w1[2 * din:2 * din + 1]
    tw = jnp.concatenate(
        [jnp.kron(jnp.eye(N, dtype=jnp.float32), c),
         jnp.tile(w1[:din], (1, N)),
         jnp.tile(b1.reshape(1, HE), (1, N))], axis=0).astype(BF)
    arrs = [w1[din:2 * din].astype(BF), tw,
            _blockdiag(w2).astype(BF),
            jnp.concatenate([b2.reshape(1, OE)] * P, axis=1)]
    for (w, bb) in p["node"]:
        arrs += [w, bb.reshape(1, -1)]
    return arrs


def kernel(x, enc_params, dec_params, lin_W, lin_b):
    f32 = jnp.float32
    e32 = jnp.kron(jnp.eye(N, dtype=f32),
                   jnp.ones((1, OE), f32)).astype(BF)
    r = jnp.arange(N)
    sall = jax.nn.one_hot(P * (r % KP) + r // KP, N, dtype=BF)
    ops = [e32, sall]
    ops += _flatten_mp(enc_params[0], D)
    ops += _flatten_mp(enc_params[1], D)
    ops += _flatten_mp(dec_params[0], LATENT)
    ops += _flatten_mp(dec_params[1], LATENT)
    ops.append(jnp.transpose(lin_W.reshape(LATENT, N, LATENT), (0, 2, 1)))
    ops.append(lin_b.reshape(N, LATENT).T)

    def const_spec(a):
        nd = a.ndim
        return pl.BlockSpec(a.shape, lambda b, _n=nd: (0,) * _n)

    in_specs = [pl.BlockSpec((2, N, D), lambda b: (b, 0, 0))]
    in_specs += [const_spec(a) for a in ops]

    latent, y = pl.pallas_call(
        _body,
        grid=(B // 2,),
        in_specs=in_specs,
        out_specs=[
            pl.BlockSpec((2, 1, LATENT), lambda b: (b, 0, 0)),
            pl.BlockSpec((2, N, D), lambda b: (b, 0, 0)),
        ],
        out_shape=[
            jax.ShapeDtypeStruct((B, 1, LATENT), f32),
            jax.ShapeDtypeStruct((B, N, D), f32),
        ],
        compiler_params=pltpu.CompilerParams(
            dimension_semantics=("parallel",)),
    )(x, *ops)
    return latent.reshape(1, B, LATENT), y


# speedup vs baseline: 3.1842x; 1.0853x over previous
import jax
import jax.numpy as jnp
from jax.experimental import pallas as pl
from jax.experimental.pallas import tpu as pltpu

N = 128
D = 16
LATENT = 8
HE = 64
OE = 32
ALPHA = 0.2
B = 32
GP = 4
P = 4
KP = N // P
PHE = P * HE
POE = P * OE
NPW = 10
BF = jnp.bfloat16


def _leaky(v):
    return jnp.maximum(v, ALPHA * v)


def _mm(a, b):
    return jax.lax.dot_general(a, b, (((a.ndim - 1,), (0,)), ((), ())),
                               preferred_element_type=jnp.float32)


def _mp_layer(x2, wb, tw, w2p, b2p, nws, e32, sall):
    def _dist_w(xg):
        sq = jnp.sum(xg * xg, axis=1, keepdims=True)
        g = jax.lax.dot_general(xg, xg, (((1,), (1,)), ((), ())),
                                preferred_element_type=jnp.float32)
        dist = sq + sq.T - 2.0 * g
        return dist, jnp.exp(-dist)

    xg = [x2[N * i:N * (i + 1)] for i in range(GP)]
    dw = [_dist_w(x) for x in xg]
    x2bf = x2.astype(BF)
    bm2 = _mm(x2bf, wb)

    def _bm4(bm2g):
        p4 = _mm(sall, bm2g.astype(BF))
        return jnp.concatenate(
            [p4[KP * m:KP * (m + 1)] for m in range(P)], axis=1).astype(BF)

    bm4 = [_bm4(bm2[N * i:N * (i + 1)]) for i in range(GP)]
    ones = jnp.ones((N, 1), BF)
    lhs = jnp.concatenate(
        [jnp.concatenate([d.astype(BF), x.astype(BF), ones], axis=1)
         for (d, _), x in zip(dw, xg)], axis=0)
    t_all = jax.lax.dot_general(lhs, tw, (((1,), (0,)), ((), ())),
                                preferred_element_type=jnp.float32)
    t_all = t_all.astype(BF)
    wcat = jnp.concatenate([w for _, w in dw], axis=0).astype(BF)
    w4_all = jax.lax.dot_general(wcat, e32, (((1,), (0,)), ((), ())),
                                 preferred_element_type=jnp.float32)
    acc = [jnp.zeros((GP * N, POE), jnp.float32) for _ in range(4)]
    for k in range(KP):
        tk = t_all[:, PHE * k:PHE * (k + 1)]
        h1k = _leaky(jnp.concatenate(
            [tk[N * i:N * (i + 1)] + bm4[i][k:k + 1, :] for i in range(GP)],
            axis=0))
        h2k = _leaky(
            jax.lax.dot_general(h1k, w2p, (((1,), (0,)), ((), ())),
                                preferred_element_type=jnp.float32) + b2p)
        acc[k % 4] = acc[k % 4] + h2k * w4_all[:, POE * k:POE * (k + 1)]
    s1 = (acc[0] + acc[1]) + (acc[2] + acc[3])
    agg = (s1[:, :OE] + s1[:, OE:2 * OE]
           + s1[:, 2 * OE:3 * OE] + s1[:, 3 * OE:])
    h = jnp.concatenate([x2, agg], axis=-1)
    for i, (nw, nb) in enumerate(nws):
        h = _mm(h, nw) + nb
        if i < len(nws) - 1:
            h = _leaky(h)
    return h


def _body(x_ref, e32_ref, sall_ref, *refs):
    refs = list(refs)
    latent_ref, y_ref = refs[-2], refs[-1]
    wrefs = refs[:-2]
    consts = (e32_ref[...], sall_ref[...])

    def mp_args(k):
        base = wrefs[k * NPW:(k + 1) * NPW]
        wb, tw, w2p, b2p = (r[...] for r in base[:4])
        nws = [(base[4 + 2 * i][...], base[5 + 2 * i][...]) for i in range(3)]
        return wb, tw, w2p, b2p, nws

    wr_ref, br_ref = wrefs[4 * NPW], wrefs[4 * NPW + 1]

    x2 = x_ref[...].reshape(GP * N, D)
    z = _mp_layer(x2, *mp_args(0), *consts)
    z = _mp_layer(z, *mp_args(1), *consts)
    lats = [jnp.sum(z[N * i:N * (i + 1)], axis=0, keepdims=True)
            for i in range(GP)]
    for i in range(GP):
        latent_ref[i] = lats[i]

    def _y0(lat):
        y0t = br_ref[...]
        for k in range(LATENT):
            y0t = y0t + wr_ref[k] * lat[0:1, k:k + 1]
        return y0t.T

    y0 = jnp.concatenate([_y0(lat) for lat in lats], axis=0)
    y = _mp_layer(y0, *mp_args(2), *consts)
    y = _mp_layer(y, *mp_args(3), *consts)
    y_ref[...] = y.reshape(GP, N, D)


def _blockdiag(m):
    z = jnp.zeros_like(m)
    return jnp.concatenate(
        [jnp.concatenate([m if mm == k else z for mm in range(P)], axis=1)
         for k in range(P)], axis=0)


def _flatten_mp(p, din):
    (w1, b1), (w2, b2) = p["edge"]
    c = w1[2 * din:2 * din + 1]
    tw = jnp.concatenate(
        [jnp.kron(jnp.eye(N, dtype=jnp.float32), c),
         jnp.tile(w1[:din], (1, N)),
         jnp.tile(b1.reshape(1, HE), (1, N))], axis=0).astype(BF)
    arrs = [w1[din:2 * din].astype(BF), tw,
            _blockdiag(w2).astype(BF),
            jnp.concatenate([b2.reshape(1, OE)] * P, axis=1)]
    for (w, bb) in p["node"]:
        arrs += [w, bb.reshape(1, -1)]
    return arrs


def kernel(x, enc_params, dec_params, lin_W, lin_b):
    f32 = jnp.float32
    e32 = jnp.kron(jnp.eye(N, dtype=f32),
                   jnp.ones((1, OE), f32)).astype(BF)
    r = jnp.arange(N)
    sall = jax.nn.one_hot(P * (r % KP) + r // KP, N, dtype=BF)
    ops = [e32, sall]
    ops += _flatten_mp(enc_params[0], D)
    ops += _flatten_mp(enc_params[1], D)
    ops += _flatten_mp(dec_params[0], LATENT)
    ops += _flatten_mp(dec_params[1], LATENT)
    ops.append(jnp.transpose(lin_W.reshape(LATENT, N, LATENT), (0, 2, 1)))
    ops.append(lin_b.reshape(N, LATENT).T)

    def const_spec(a):
        nd = a.ndim
        return pl.BlockSpec(a.shape, lambda b, _n=nd: (0,) * _n)

    in_specs = [pl.BlockSpec((GP, N, D), lambda b: (b, 0, 0))]
    in_specs += [const_spec(a) for a in ops]

    latent, y = pl.pallas_call(
        _body,
        grid=(B // GP,),
        in_specs=in_specs,
        out_specs=[
            pl.BlockSpec((GP, 1, LATENT), lambda b: (b, 0, 0)),
            pl.BlockSpec((GP, N, D), lambda b: (b, 0, 0)),
        ],
        out_shape=[
            jax.ShapeDtypeStruct((B, 1, LATENT), f32),
            jax.ShapeDtypeStruct((B, N, D), f32),
        ],
        compiler_params=pltpu.CompilerParams(
            dimension_semantics=("parallel",)),
    )(x, *ops)
    return latent.reshape(1, B, LATENT), y


# speedup vs baseline: 3.1866x; 1.0008x over previous
import jax
import jax.numpy as jnp
from jax.experimental import pallas as pl
from jax.experimental.pallas import tpu as pltpu

N = 128
D = 16
LATENT = 8
HE = 64
OE = 32
ALPHA = 0.2
B = 32
GP = 4
P = 4
KP = N // P
PHE = P * HE
POE = P * OE
NPW = 10
BF = jnp.bfloat16


def _leaky(v):
    return jnp.maximum(v, ALPHA * v)


def _mm(a, b):
    return jax.lax.dot_general(a, b, (((a.ndim - 1,), (0,)), ((), ())),
                               preferred_element_type=jnp.float32)


def _mp_layer(x2, wb, tw, w2p, b2p, nws, e32, sall):
    def _dist_w(xg):
        sq = jnp.sum(xg * xg, axis=1, keepdims=True)
        g = jax.lax.dot_general(xg, xg, (((1,), (1,)), ((), ())),
                                preferred_element_type=jnp.float32)
        dist = sq + sq.T - 2.0 * g
        return dist, jnp.exp(-dist)

    xg = [x2[N * i:N * (i + 1)] for i in range(GP)]
    dw = [_dist_w(x) for x in xg]
    x2bf = x2.astype(BF)
    bm2 = _mm(x2bf, wb)

    def _bm4(bm2g):
        p4 = _mm(sall, bm2g.astype(BF))
        return jnp.concatenate(
            [p4[KP * m:KP * (m + 1)] for m in range(P)], axis=1).astype(BF)

    bm4 = [_bm4(bm2[N * i:N * (i + 1)]) for i in range(GP)]
    ones = jnp.ones((N, 1), BF)
    lhs = jnp.concatenate(
        [jnp.concatenate([d.astype(BF), x.astype(BF), ones], axis=1)
         for (d, _), x in zip(dw, xg)], axis=0)
    t_all = jax.lax.dot_general(lhs, tw, (((1,), (0,)), ((), ())),
                                preferred_element_type=jnp.float32)
    wcat = jnp.concatenate([w for _, w in dw], axis=0).astype(BF)
    w4_all = jax.lax.dot_general(wcat, e32, (((1,), (0,)), ((), ())),
                                 preferred_element_type=jnp.float32)
    acc = [jnp.zeros((GP * N, POE), jnp.float32) for _ in range(4)]
    for k in range(KP):
        tk = t_all[:, PHE * k:PHE * (k + 1)].astype(BF)
        h1k = _leaky(jnp.concatenate(
            [tk[N * i:N * (i + 1)] + bm4[i][k:k + 1, :] for i in range(GP)],
            axis=0))
        h2k = _leaky(
            jax.lax.dot_general(h1k, w2p, (((1,), (0,)), ((), ())),
                                preferred_element_type=jnp.float32) + b2p)
        acc[k % 4] = acc[k % 4] + h2k * w4_all[:, POE * k:POE * (k + 1)]
    s1 = (acc[0] + acc[1]) + (acc[2] + acc[3])
    agg = (s1[:, :OE] + s1[:, OE:2 * OE]
           + s1[:, 2 * OE:3 * OE] + s1[:, 3 * OE:])
    h = jnp.concatenate([x2, agg], axis=-1)
    for i, (nw, nb) in enumerate(nws):
        h = _mm(h, nw) + nb
        if i < len(nws) - 1:
            h = _leaky(h)
    return h


def _body(x_ref, e32_ref, sall_ref, *refs):
    refs = list(refs)
    latent_ref, y_ref = refs[-2], refs[-1]
    wrefs = refs[:-2]
    consts = (e32_ref[...], sall_ref[...])

    def mp_args(k):
        base = wrefs[k * NPW:(k + 1) * NPW]
        wb, tw, w2p, b2p = (r[...] for r in base[:4])
        nws = [(base[4 + 2 * i][...], base[5 + 2 * i][...]) for i in range(3)]
        return wb, tw, w2p, b2p, nws

    wr_ref, br_ref = wrefs[4 * NPW], wrefs[4 * NPW + 1]

    x2 = x_ref[...].reshape(GP * N, D)
    z = _mp_layer(x2, *mp_args(0), *consts)
    z = _mp_layer(z, *mp_args(1), *consts)
    lats = [jnp.sum(z[N * i:N * (i + 1)], axis=0, keepdims=True)
            for i in range(GP)]
    for i in range(GP):
        latent_ref[i] = lats[i]

    def _y0(lat):
        y0t = br_ref[...]
        for k in range(LATENT):
            y0t = y0t + wr_ref[k] * lat[0:1, k:k + 1]
        return y0t.T

    y0 = jnp.concatenate([_y0(lat) for lat in lats], axis=0)
    y = _mp_layer(y0, *mp_args(2), *consts)
    y = _mp_layer(y, *mp_args(3), *consts)
    y_ref[...] = y.reshape(GP, N, D)


def _blockdiag(m):
    z = jnp.zeros_like(m)
    return jnp.concatenate(
        [jnp.concatenate([m if mm == k else z for mm in range(P)], axis=1)
         for k in range(P)], axis=0)


def _flatten_mp(p, din):
    (w1, b1), (w2, b2) = p["edge"]
    c = w1[2 * din:2 * din + 1]
    tw = jnp.concatenate(
        [jnp.kron(jnp.eye(N, dtype=jnp.float32), c),
         jnp.tile(w1[:din], (1, N)),
         jnp.tile(b1.reshape(1, HE), (1, N))], axis=0).astype(BF)
    arrs = [w1[din:2 * din].astype(BF), tw,
            _blockdiag(w2).astype(BF),
            jnp.concatenate([b2.reshape(1, OE)] * P, axis=1)]
    for (w, bb) in p["node"]:
        arrs += [w, bb.reshape(1, -1)]
    return arrs


def kernel(x, enc_params, dec_params, lin_W, lin_b):
    f32 = jnp.float32
    e32 = jnp.kron(jnp.eye(N, dtype=f32),
                   jnp.ones((1, OE), f32)).astype(BF)
    r = jnp.arange(N)
    sall = jax.nn.one_hot(P * (r % KP) + r // KP, N, dtype=BF)
    ops = [e32, sall]
    ops += _flatten_mp(enc_params[0], D)
    ops += _flatten_mp(enc_params[1], D)
    ops += _flatten_mp(dec_params[0], LATENT)
    ops += _flatten_mp(dec_params[1], LATENT)
    ops.append(jnp.transpose(lin_W.reshape(LATENT, N, LATENT), (0, 2, 1)))
    ops.append(lin_b.reshape(N, LATENT).T)

    def const_spec(a):
        nd = a.ndim
        return pl.BlockSpec(a.shape, lambda b, _n=nd: (0,) * _n)

    in_specs = [pl.BlockSpec((GP, N, D), lambda b: (b, 0, 0))]
    in_specs += [const_spec(a) for a in ops]

    latent, y = pl.pallas_call(
        _body,
        grid=(B // GP,),
        in_specs=in_specs,
        out_specs=[
            pl.BlockSpec((GP, 1, LATENT), lambda b: (b, 0, 0)),
            pl.BlockSpec((GP, N, D), lambda b: (b, 0, 0)),
        ],
        out_shape=[
            jax.ShapeDtypeStruct((B, 1, LATENT), f32),
            jax.ShapeDtypeStruct((B, N, D), f32),
        ],
        compiler_params=pltpu.CompilerParams(
            dimension_semantics=("parallel",)),
    )(x, *ops)
    return latent.reshape(1, B, LATENT), y
